# Initial kernel scaffold; baseline (speedup 1.0000x reference)
import jax
import jax.numpy as jnp
from jax.experimental import pallas as pl


def kernel(nodes, neighbor_indices, params):
    raise NotImplementedError("write your pallas kernel here")


# speedup vs baseline: 2.6240x; 2.6240x over previous
import functools

import jax
import jax.numpy as jnp
from jax import lax
from jax.experimental import pallas as pl
from jax.experimental.pallas import tpu as pltpu
from jax.experimental.pallas import tpu_sc as plsc

NN = 50000
D = 256
K = 6

NC, NS = 2, 16
NW = NC * NS
CPW = 1600
NPAD = NW * CPW
B = 32
NB = CPW // B

_MESH = plsc.VectorSubcoreMesh(
    core_axis_name="c", subcore_axis_name="s", num_cores=NC, num_subcores=NS
)


@functools.partial(
    pl.kernel,
    out_type=jax.ShapeDtypeStruct((NPAD, D), jnp.float32),
    mesh=_MESH,
    scratch_types=[
        pltpu.VMEM((K, CPW), jnp.int32),
        pltpu.VMEM((2, K, B, D), jnp.float32),
        pltpu.SemaphoreType.DMA,
        pltpu.SemaphoreType.DMA,
    ],
)
def _sc_neighbor_sum(nodes_hbm, idxt_hbm, out_hbm, idx_v, bufs, gsem0, gsem1):
    wid = lax.axis_index("s") * NC + lax.axis_index("c")
    base = wid * CPW
    pltpu.sync_copy(idxt_hbm.at[wid], idx_v)
    gsems = (gsem0, gsem1)

    def fire(b, s):
        off = b * B
        for k in range(K):
            pltpu.make_async_copy(
                nodes_hbm.at[idx_v.at[k, pl.ds(off, B)]],
                bufs.at[s, k],
                gsems[s],
            ).start()

    def wait_set(s):
        for k in range(K):
            pltpu.make_async_copy(
                nodes_hbm.at[idx_v.at[0, pl.ds(0, B)]],
                bufs.at[s, k],
                gsems[s],
            ).wait()

    def reduce_set(s):
        def row_body(r, carry):
            for c in range(D // 16):
                dsl = pl.ds(c * 16, 16)
                v = bufs[s, 0, r, dsl]
                for k in range(1, K):
                    v = v + bufs[s, k, r, dsl]
                bufs[s, 0, r, dsl] = v * (1.0 / K)
            return carry
        lax.fori_loop(0, B, row_body, 0)

    def flush(b, s):
        pltpu.sync_copy(bufs.at[s, 0], out_hbm.at[pl.ds(base + b * B, B)])

    fire(0, 0)

    def body2(i, carry):
        b0 = i * 2
        for s in (0, 1):
            b = b0 + s

            @pl.when(b + 1 < NB)
            def _():
                fire(b + 1, 1 - s)

            wait_set(s)
            reduce_set(s)
            flush(b, s)
        return carry

    lax.fori_loop(0, NB // 2, body2, 0)


RB = 400


def _tc_layer_body(x_ref, nm_ref, ws_ref, wn_ref, b_ref, sc_ref, of_ref, o_ref):
    x = x_ref[...]
    h = x + b_ref[...]
    h = h + jnp.dot(x, ws_ref[...], preferred_element_type=jnp.float32)
    h = h + jnp.dot(nm_ref[...], wn_ref[...], preferred_element_type=jnp.float32)
    mu = jnp.mean(h, axis=-1, keepdims=True)
    xc = h - mu
    var = jnp.mean(xc * xc, axis=-1, keepdims=True)
    y = xc * lax.rsqrt(var + 1e-5) * sc_ref[...] + of_ref[...]
    o_ref[...] = jnp.maximum(y, 0.0)


def _tc_layer(x, nm, ws, wn, b, scale, offset):
    return pl.pallas_call(
        _tc_layer_body,
        grid=(NN // RB,),
        in_specs=[
            pl.BlockSpec((RB, D), lambda i: (i, 0)),
            pl.BlockSpec((RB, D), lambda i: (i, 0)),
            pl.BlockSpec((D, D), lambda i: (0, 0)),
            pl.BlockSpec((D, D), lambda i: (0, 0)),
            pl.BlockSpec((1, D), lambda i: (0, 0)),
            pl.BlockSpec((1, D), lambda i: (0, 0)),
            pl.BlockSpec((1, D), lambda i: (0, 0)),
        ],
        out_specs=pl.BlockSpec((RB, D), lambda i: (i, 0)),
        out_shape=jax.ShapeDtypeStruct((NN, D), jnp.float32),
    )(x, nm, ws, wn, b, scale, offset)


def kernel(nodes, neighbor_indices, params):
    idx_pad = jnp.zeros((NPAD, K), jnp.int32).at[:NN].set(neighbor_indices)
    idxt = idx_pad.reshape(NW, CPW, K).transpose(0, 2, 1)
    x = nodes
    for p in params:
        nm = _sc_neighbor_sum(x, idxt)
        b = (p["bs"] + p["bn"]).reshape(1, D)
        x = _tc_layer(x, nm, p["Ws"], p["Wn"], b,
                      p["scale"].reshape(1, D), p["offset"].reshape(1, D))
    return x


# speedup vs baseline: 3.3544x; 1.2783x over previous
import functools

import jax
import jax.numpy as jnp
from jax import lax
from jax.experimental import pallas as pl
from jax.experimental.pallas import tpu as pltpu
from jax.experimental.pallas import tpu_sc as plsc

NN = 50000
D = 256
DH = D // 2
K = 6

NC, NS = 2, 16
NW = NC * NS
CPW = 1600
NPAD = NW * CPW
B = 32
NB = CPW // B

_MESH = plsc.VectorSubcoreMesh(
    core_axis_name="c", subcore_axis_name="s", num_cores=NC, num_subcores=NS
)


@functools.partial(
    pl.kernel,
    out_type=jax.ShapeDtypeStruct((NPAD, D), jnp.float32),
    mesh=_MESH,
    scratch_types=[
        pltpu.VMEM((K, CPW), jnp.int32),
        pltpu.VMEM((2, K, B, DH), jnp.int32),
        pltpu.VMEM((B, D), jnp.float32),
        pltpu.SemaphoreType.DMA,
        pltpu.SemaphoreType.DMA,
    ],
)
def _sc_neighbor_mean(tab_hbm, idxt_hbm, out_hbm, idx_v, bufs, acc, gsem0, gsem1):
    wid = lax.axis_index("s") * NC + lax.axis_index("c")
    base = wid * CPW
    pltpu.sync_copy(idxt_hbm.at[wid], idx_v)
    gsems = (gsem0, gsem1)

    def fire(b, s):
        off = b * B
        for k in range(K):
            pltpu.make_async_copy(
                tab_hbm.at[idx_v.at[k, pl.ds(off, B)]],
                bufs.at[s, k],
                gsems[s],
            ).start()

    def wait_set(s):
        for k in range(K):
            pltpu.make_async_copy(
                tab_hbm.at[idx_v.at[0, pl.ds(0, B)]],
                bufs.at[s, k],
                gsems[s],
            ).wait()

    def reduce_set(s):
        def row_body(r, carry):
            for c in range(DH // 16):
                dsl = pl.ds(c * 16, 16)
                v = bufs[s, 0, r, dsl]
                a = lax.bitcast_convert_type(v << 16, jnp.float32)
                b = lax.bitcast_convert_type(v & (-65536), jnp.float32)
                for k in range(1, K):
                    v = bufs[s, k, r, dsl]
                    a = a + lax.bitcast_convert_type(v << 16, jnp.float32)
                    b = b + lax.bitcast_convert_type(v & (-65536), jnp.float32)
                acc[r, pl.ds(c * 16, 16)] = a * (1.0 / K)
                acc[r, pl.ds(DH + c * 16, 16)] = b * (1.0 / K)
            return carry
        lax.fori_loop(0, B, row_body, 0)

    def flush(b):
        pltpu.sync_copy(acc, out_hbm.at[pl.ds(base + b * B, B)])

    fire(0, 0)

    def body2(i, carry):
        b0 = i * 2
        for s in (0, 1):
            b = b0 + s

            @pl.when(b + 1 < NB)
            def _():
                fire(b + 1, 1 - s)

            wait_set(s)
            reduce_set(s)
            flush(b)
        return carry

    lax.fori_loop(0, NB // 2, body2, 0)


RB = 400

def _pack_rows(y):
    u = lax.bitcast_convert_type(y, jnp.uint32)

    def rne(t):
        return (t + 0x7FFF + ((t >> 16) & 1)) >> 16

    p = (rne(u[:, DH:]) << 16) | rne(u[:, :DH])
    return lax.bitcast_convert_type(p, jnp.int32)


def _tc_layer_body(x_ref, nm_ref, ws_ref, wn_ref, b_ref, sc_ref, of_ref,
                   o_ref, p_ref):
    x = x_ref[...]
    h = x + b_ref[...]
    h = h + jnp.dot(x, ws_ref[...], preferred_element_type=jnp.float32)
    h = h + jnp.dot(nm_ref[...], wn_ref[...], preferred_element_type=jnp.float32)
    mu = jnp.mean(h, axis=-1, keepdims=True)
    xc = h - mu
    var = jnp.mean(xc * xc, axis=-1, keepdims=True)
    y = xc * lax.rsqrt(var + 1e-5) * sc_ref[...] + of_ref[...]
    y = jnp.maximum(y, 0.0)
    o_ref[...] = y
    p_ref[...] = _pack_rows(y)


def _tc_layer(x, nm, ws, wn, b, scale, offset):
    return pl.pallas_call(
        _tc_layer_body,
        grid=(NN // RB,),
        in_specs=[
            pl.BlockSpec((RB, D), lambda i: (i, 0)),
            pl.BlockSpec((RB, D), lambda i: (i, 0)),
            pl.BlockSpec((D, D), lambda i: (0, 0)),
            pl.BlockSpec((D, D), lambda i: (0, 0)),
            pl.BlockSpec((1, D), lambda i: (0, 0)),
            pl.BlockSpec((1, D), lambda i: (0, 0)),
            pl.BlockSpec((1, D), lambda i: (0, 0)),
        ],
        out_specs=[
            pl.BlockSpec((RB, D), lambda i: (i, 0)),
            pl.BlockSpec((RB, DH), lambda i: (i, 0)),
        ],
        out_shape=[
            jax.ShapeDtypeStruct((NN, D), jnp.float32),
            jax.ShapeDtypeStruct((NN, DH), jnp.int32),
        ],
    )(x, nm, ws, wn, b, scale, offset)


def kernel(nodes, neighbor_indices, params):
    idx_pad = jnp.zeros((NPAD, K), jnp.int32).at[:NN].set(neighbor_indices)
    idxt = idx_pad.reshape(NW, CPW, K).transpose(0, 2, 1)
    x = nodes
    ptab = _pack_rows(nodes)
    for p in params:
        nm = _sc_neighbor_mean(ptab, idxt)
        b = (p["bs"] + p["bn"]).reshape(1, D)
        x, ptab = _tc_layer(x, nm, p["Ws"], p["Wn"], b,
                            p["scale"].reshape(1, D), p["offset"].reshape(1, D))
    return x


# speedup vs baseline: 3.4002x; 1.0136x over previous
import functools

import jax
import jax.numpy as jnp
from jax import lax
from jax.experimental import pallas as pl
from jax.experimental.pallas import tpu as pltpu
from jax.experimental.pallas import tpu_sc as plsc

NN = 50000
D = 256
DH = D // 2
K = 6

NC, NS = 2, 16
NW = NC * NS
CPW = 1600
NPAD = NW * CPW
B = 64
NB = CPW // B

_MESH = plsc.VectorSubcoreMesh(
    core_axis_name="c", subcore_axis_name="s", num_cores=NC, num_subcores=NS
)


@functools.partial(
    pl.kernel,
    out_type=jax.ShapeDtypeStruct((NPAD, D), jnp.float32),
    mesh=_MESH,
    scratch_types=[
        pltpu.VMEM((K, CPW), jnp.int32),
        pltpu.VMEM((2, K, B, DH), jnp.int32),
        pltpu.VMEM((B, D), jnp.float32),
        pltpu.SemaphoreType.DMA,
        pltpu.SemaphoreType.DMA,
    ],
)
def _sc_neighbor_mean(tab_hbm, idxt_hbm, out_hbm, idx_v, bufs, acc, gsem0, gsem1):
    wid = lax.axis_index("s") * NC + lax.axis_index("c")
    base = wid * CPW
    pltpu.sync_copy(idxt_hbm.at[wid], idx_v)
    gsems = (gsem0, gsem1)

    def fire(b, s):
        off = b * B
        for k in range(K):
            pltpu.make_async_copy(
                tab_hbm.at[idx_v.at[k, pl.ds(off, B)]],
                bufs.at[s, k],
                gsems[s],
            ).start()

    def wait_set(s):
        for k in range(K):
            pltpu.make_async_copy(
                tab_hbm.at[idx_v.at[0, pl.ds(0, B)]],
                bufs.at[s, k],
                gsems[s],
            ).wait()

    def reduce_set(s):
        def row_body(r, carry):
            for c in range(DH // 16):
                dsl = pl.ds(c * 16, 16)
                v = bufs[s, 0, r, dsl]
                a = lax.bitcast_convert_type(v << 16, jnp.float32)
                b = lax.bitcast_convert_type(v & (-65536), jnp.float32)
                for k in range(1, K):
                    v = bufs[s, k, r, dsl]
                    a = a + lax.bitcast_convert_type(v << 16, jnp.float32)
                    b = b + lax.bitcast_convert_type(v & (-65536), jnp.float32)
                acc[r, pl.ds(c * 16, 16)] = a * (1.0 / K)
                acc[r, pl.ds(DH + c * 16, 16)] = b * (1.0 / K)
            return carry
        lax.fori_loop(0, B, row_body, 0)

    def flush(b):
        pltpu.sync_copy(acc, out_hbm.at[pl.ds(base + b * B, B)])

    fire(0, 0)

    def body2(i, carry):
        b0 = i * 2
        for s in (0, 1):
            b = b0 + s

            @pl.when(b + 1 < NB)
            def _():
                fire(b + 1, 1 - s)

            wait_set(s)
            reduce_set(s)
            flush(b)
        return carry

    lax.fori_loop(0, NB // 2, body2, 0)
    wait_set(0)
    reduce_set(0)
    flush(NB - 1)


RB = 400

def _pack_rows(y):
    u = lax.bitcast_convert_type(y, jnp.uint32)

    def rne(t):
        return (t + 0x7FFF + ((t >> 16) & 1)) >> 16

    p = (rne(u[:, DH:]) << 16) | rne(u[:, :DH])
    return lax.bitcast_convert_type(p, jnp.int32)


def _tc_layer_body(x_ref, nm_ref, ws_ref, wn_ref, b_ref, sc_ref, of_ref,
                   o_ref, p_ref):
    x = x_ref[...]
    h = x + b_ref[...]
    h = h + jnp.dot(x, ws_ref[...], preferred_element_type=jnp.float32)
    h = h + jnp.dot(nm_ref[...], wn_ref[...], preferred_element_type=jnp.float32)
    mu = jnp.mean(h, axis=-1, keepdims=True)
    xc = h - mu
    var = jnp.mean(xc * xc, axis=-1, keepdims=True)
    y = xc * lax.rsqrt(var + 1e-5) * sc_ref[...] + of_ref[...]
    y = jnp.maximum(y, 0.0)
    o_ref[...] = y
    p_ref[...] = _pack_rows(y)


def _tc_layer(x, nm, ws, wn, b, scale, offset):
    return pl.pallas_call(
        _tc_layer_body,
        grid=(NN // RB,),
        in_specs=[
            pl.BlockSpec((RB, D), lambda i: (i, 0)),
            pl.BlockSpec((RB, D), lambda i: (i, 0)),
            pl.BlockSpec((D, D), lambda i: (0, 0)),
            pl.BlockSpec((D, D), lambda i: (0, 0)),
            pl.BlockSpec((1, D), lambda i: (0, 0)),
            pl.BlockSpec((1, D), lambda i: (0, 0)),
            pl.BlockSpec((1, D), lambda i: (0, 0)),
        ],
        out_specs=[
            pl.BlockSpec((RB, D), lambda i: (i, 0)),
            pl.BlockSpec((RB, DH), lambda i: (i, 0)),
        ],
        out_shape=[
            jax.ShapeDtypeStruct((NN, D), jnp.float32),
            jax.ShapeDtypeStruct((NN, DH), jnp.int32),
        ],
    )(x, nm, ws, wn, b, scale, offset)


def kernel(nodes, neighbor_indices, params):
    idx_pad = jnp.zeros((NPAD, K), jnp.int32).at[:NN].set(neighbor_indices)
    idxt = idx_pad.reshape(NW, CPW, K).transpose(0, 2, 1)
    x = nodes
    ptab = _pack_rows(nodes)
    for p in params:
        nm = _sc_neighbor_mean(ptab, idxt)
        b = (p["bs"] + p["bn"]).reshape(1, D)
        x, ptab = _tc_layer(x, nm, p["Ws"], p["Wn"], b,
                            p["scale"].reshape(1, D), p["offset"].reshape(1, D))
    return x
